# baseline (speedup 1.0000x reference)
import jax
import jax.numpy as jnp
from jax import lax
from jax.experimental import pallas as pl
from jax.experimental.pallas import tpu as pltpu
from jax.experimental.pallas import tpu_sc as plsc

N_FACTORS = 64
BATCH = 16384
NC = 2
NS = 16
NW = NC * NS
B_PER_W = BATCH // NW
PHASE = 128
N_PHASES = B_PER_W // PHASE
G_PER_PHASE = PHASE // 16


def _sc_body(users_hbm, items_hbm, uf_hbm, if_hbm, out_hbm,
             idx_u, idx_v, u_a, u_b, v_a, v_b, out_buf, sem_u, sem_v):
    wid = lax.axis_index("s") * NC + lax.axis_index("c")
    base = wid * B_PER_W

    pltpu.sync_copy(users_hbm.at[pl.ds(base, B_PER_W)], idx_u)
    pltpu.sync_copy(items_hbm.at[pl.ds(base, B_PER_W)], idx_v)

    u_bufs = [u_a, u_b]
    v_bufs = [v_a, v_b]

    def fire(ph):
        ub, vb = u_bufs[ph % 2], v_bufs[ph % 2]

        def f(g, _):
            s = ph * PHASE + g * 16
            iu = lax.shift_right_logical(idx_u[pl.ds(s, 16)], 1)
            iv = lax.shift_right_logical(idx_v[pl.ds(s, 16)], 1)
            dst = pl.ds(g * 16, 16)
            pltpu.async_copy(uf_hbm.at[iu], ub.at[dst], sem_u)
            pltpu.async_copy(if_hbm.at[iv], vb.at[dst], sem_v)
            return 0

        lax.fori_loop(0, G_PER_PHASE, f, 0)

    def drain(ph):
        pltpu.make_async_copy(
            uf_hbm.at[pl.ds(0, PHASE)], u_bufs[ph % 2], sem_u).wait()
        pltpu.make_async_copy(
            if_hbm.at[pl.ds(0, PHASE)], v_bufs[ph % 2], sem_v).wait()

    fire(0)

    lane = lax.iota(jnp.int32, 16)
    last = lane == 15

    def compute(ph):
        ub, vb = u_bufs[ph % 2], v_bufs[ph % 2]

        def row_body(b, _):
            gb = ph * PHASE + b
            cbase = jnp.bitwise_and(gb, ~15)
            lidx = jnp.full((16,), jnp.bitwise_and(gb, 15), jnp.int32)
            pu = jnp.bitwise_and(idx_u[pl.ds(cbase, 16)], 1)
            pv = jnp.bitwise_and(idx_v[pl.ds(cbase, 16)], 1)
            hu = pu[lidx] == 1
            hv = pv[lidx] == 1
            acc = jnp.zeros((16,), jnp.float32)
            for c in range(4):
                ulo = ub[b, pl.ds(c * 16, 16)]
                uhi = ub[b, pl.ds(64 + c * 16, 16)]
                vlo = vb[b, pl.ds(c * 16, 16)]
                vhi = vb[b, pl.ds(64 + c * 16, 16)]
                uu = jnp.where(hu, uhi, ulo)
                vv = jnp.where(hv, vhi, vlo)
                acc = acc + uu * vv
            s = lax.cumsum(acc)
            plsc.store_scatter(out_buf, [jnp.zeros((16,), jnp.int32) + gb],
                               s, mask=last)
            return 0

        lax.fori_loop(0, PHASE, row_body, 0)

    for ph in range(N_PHASES):
        drain(ph)
        if ph + 1 < N_PHASES:
            fire(ph + 1)
        compute(ph)

    pltpu.sync_copy(out_buf, out_hbm.at[pl.ds(base, B_PER_W)])


@jax.jit
def _mf_dot(users, items, uf2, if2):
    mesh = plsc.VectorSubcoreMesh(
        core_axis_name="c", subcore_axis_name="s",
        num_cores=NC, num_subcores=NS)
    k = pl.kernel(
        _sc_body,
        out_type=jax.ShapeDtypeStruct((BATCH,), jnp.float32),
        mesh=mesh,
        compiler_params=pltpu.CompilerParams(
            needs_layout_passes=False, use_tc_tiling_on_sc=True,
            disable_bounds_checks=True),
        scratch_types=[
            pltpu.VMEM((B_PER_W,), jnp.int32),
            pltpu.VMEM((B_PER_W,), jnp.int32),
            pltpu.VMEM((PHASE, 2 * N_FACTORS), jnp.float32),
            pltpu.VMEM((PHASE, 2 * N_FACTORS), jnp.float32),
            pltpu.VMEM((PHASE, 2 * N_FACTORS), jnp.float32),
            pltpu.VMEM((PHASE, 2 * N_FACTORS), jnp.float32),
            pltpu.VMEM((B_PER_W,), jnp.float32),
            pltpu.SemaphoreType.DMA,
            pltpu.SemaphoreType.DMA,
        ],
    )
    return k(users, items, uf2, if2)


def kernel(data, user_factors, item_factors):
    users = data[:, 0].astype(jnp.int32)
    items = data[:, 1].astype(jnp.int32)
    uf2 = user_factors.reshape(50000, 2 * N_FACTORS)
    if2 = item_factors.reshape(50000, 2 * N_FACTORS)
    return _mf_dot(users, items, uf2, if2)


# speedup vs baseline: 1.0122x; 1.0122x over previous
import jax
import jax.numpy as jnp
from jax import lax
from jax.experimental import pallas as pl
from jax.experimental.pallas import tpu as pltpu
from jax.experimental.pallas import tpu_sc as plsc

N_FACTORS = 64
BATCH = 16384
NC = 2
NS = 16
NW = NC * NS
B_PER_W = BATCH // NW
GROUPS = B_PER_W // 16


def _sc_body(users_hbm, items_hbm, uf_hbm, if_hbm, out_hbm,
             idx_u, idx_v, u_rows, v_rows, out_buf, sem_u, sem_v):
    wid = lax.axis_index("s") * NC + lax.axis_index("c")
    base = wid * B_PER_W

    pltpu.sync_copy(users_hbm.at[pl.ds(base, B_PER_W)], idx_u)
    pltpu.sync_copy(items_hbm.at[pl.ds(base, B_PER_W)], idx_v)

    def fire(g, _):
        iu = idx_u[pl.ds(g * 16, 16)]
        iv = idx_v[pl.ds(g * 16, 16)]
        dst = pl.ds(g * 16, 16)
        pltpu.async_copy(uf_hbm.at[iu], u_rows.at[dst], sem_u)
        pltpu.async_copy(if_hbm.at[iv], v_rows.at[dst], sem_v)
        return 0

    lax.fori_loop(0, GROUPS, fire, 0)

    pltpu.make_async_copy(uf_hbm.at[pl.ds(0, B_PER_W)], u_rows, sem_u).wait()
    pltpu.make_async_copy(if_hbm.at[pl.ds(0, B_PER_W)], v_rows, sem_v).wait()

    lane = lax.iota(jnp.int32, 16)
    last = lane == 15

    def row_body(b, _):
        u0 = u_rows[b, pl.ds(0, 16)]
        u1 = u_rows[b, pl.ds(16, 16)]
        u2 = u_rows[b, pl.ds(32, 16)]
        u3 = u_rows[b, pl.ds(48, 16)]
        v0 = v_rows[b, pl.ds(0, 16)]
        v1 = v_rows[b, pl.ds(16, 16)]
        v2 = v_rows[b, pl.ds(32, 16)]
        v3 = v_rows[b, pl.ds(48, 16)]
        p = (u0 * v0 + u1 * v1) + (u2 * v2 + u3 * v3)
        s = lax.cumsum(p)
        plsc.store_scatter(out_buf, [jnp.zeros((16,), jnp.int32) + b], s,
                           mask=last)
        return 0

    lax.fori_loop(0, B_PER_W, row_body, 0)

    pltpu.sync_copy(out_buf, out_hbm.at[pl.ds(base, B_PER_W)])


@jax.jit
def _mf_dot(users, items, user_factors, item_factors):
    mesh = plsc.VectorSubcoreMesh(
        core_axis_name="c", subcore_axis_name="s",
        num_cores=NC, num_subcores=NS)
    k = pl.kernel(
        _sc_body,
        out_type=jax.ShapeDtypeStruct((BATCH,), jnp.float32),
        mesh=mesh,
        compiler_params=pltpu.CompilerParams(
            needs_layout_passes=False, use_tc_tiling_on_sc=False,
            disable_bounds_checks=True),
        scratch_types=[
            pltpu.VMEM((B_PER_W,), jnp.int32),
            pltpu.VMEM((B_PER_W,), jnp.int32),
            pltpu.VMEM((B_PER_W, N_FACTORS), jnp.float32),
            pltpu.VMEM((B_PER_W, N_FACTORS), jnp.float32),
            pltpu.VMEM((B_PER_W,), jnp.float32),
            pltpu.SemaphoreType.DMA,
            pltpu.SemaphoreType.DMA,
        ],
    )
    return k(users, items, user_factors, item_factors)


def kernel(data, user_factors, item_factors):
    users = data[:, 0].astype(jnp.int32)
    items = data[:, 1].astype(jnp.int32)
    return _mf_dot(users, items, user_factors, item_factors)
